# baseline (speedup 1.0000x reference)
import functools

import jax
import jax.numpy as jnp
from jax import lax
from jax.experimental import pallas as pl
from jax.experimental.pallas import tpu as pltpu
from jax.experimental.pallas import tpu_sc as plsc

BATCH = 16384
EMBED = 32
_INFO = plsc.get_sparse_core_info()
NC, NS, L = _INFO.num_cores, _INFO.num_subcores, _INFO.num_lanes
NW = NC * NS
B_PER_W = BATCH // NW
N_CHUNK = 4
CHUNK = B_PER_W // N_CHUNK


def _tec_body(users_hbm, autos_hbm, table_u_hbm, table_a_hbm, w_hbm, b_hbm,
              out_hbm, idx_u, idx_a, rows_u, rows_a, out_v, w_v, b_v, sem):
    wid = lax.axis_index("s") * NC + lax.axis_index("c")

    pltpu.sync_copy(users_hbm.at[wid], idx_u)
    pltpu.sync_copy(autos_hbm.at[wid], idx_a)
    pltpu.sync_copy(w_hbm, w_v)
    pltpu.sync_copy(b_hbm, b_v)

    copies = []
    for j in range(N_CHUNK):
        copies.append(pltpu.async_copy(
            table_u_hbm.at[idx_u.at[j]],
            rows_u.at[pl.ds(j * CHUNK, CHUNK)], sem))
        copies.append(pltpu.async_copy(
            table_a_hbm.at[idx_a.at[j]],
            rows_a.at[pl.ds(j * CHUNK, CHUNK)], sem))
    for c in copies:
        c.wait()

    lane = lax.broadcasted_iota(jnp.int32, (L,), 0)
    bvec = b_v[...]

    def body(g, carry):
        base = g * L
        ridx = lane + base
        acc = bvec
        for d in range(EMBED):
            cu = plsc.load_gather(rows_u, [ridx, lane * 0 + d])
            acc = acc + cu * w_v[d, :]
        for d in range(EMBED):
            ca = plsc.load_gather(rows_a, [ridx, lane * 0 + d])
            acc = acc + ca * w_v[EMBED + d, :]
        out_v[pl.ds(base, L)] = acc
        return carry

    lax.fori_loop(0, B_PER_W // L, body, 0)

    pltpu.sync_copy(out_v, out_hbm.at[pl.ds(wid * B_PER_W, B_PER_W)])


@jax.jit
def _run(users_r, autos_r, user_table, auto_table, w64, b16):
    mesh = plsc.VectorSubcoreMesh(core_axis_name="c", subcore_axis_name="s")
    f = functools.partial(
        pl.kernel, mesh=mesh,
        compiler_params=pltpu.CompilerParams(needs_layout_passes=False,
                                             use_tc_tiling_on_sc=False),
        out_type=jax.ShapeDtypeStruct((BATCH,), jnp.float32),
        scratch_types=[
            pltpu.VMEM((N_CHUNK, CHUNK), jnp.int32),
            pltpu.VMEM((N_CHUNK, CHUNK), jnp.int32),
            pltpu.VMEM((B_PER_W, EMBED), jnp.float32),
            pltpu.VMEM((B_PER_W, EMBED), jnp.float32),
            pltpu.VMEM((B_PER_W,), jnp.float32),
            pltpu.VMEM((2 * EMBED, L), jnp.float32),
            pltpu.VMEM((L,), jnp.float32),
            pltpu.SemaphoreType.DMA,
        ],
    )(_tec_body)
    return f(users_r, autos_r, user_table, auto_table, w64, b16)


def kernel(users, autos, user_table, auto_table, W, b):
    users_r = users.astype(jnp.int32).reshape(NW, N_CHUNK, CHUNK)
    autos_r = autos.astype(jnp.int32).reshape(NW, N_CHUNK, CHUNK)
    w64 = jnp.broadcast_to(W.astype(jnp.float32).reshape(2 * EMBED, 1),
                           (2 * EMBED, L))
    b16 = jnp.broadcast_to(b.astype(jnp.float32), (L,))
    out = _run(users_r, autos_r, user_table, auto_table, w64, b16)
    return out.reshape(BATCH, 1)


# speedup vs baseline: 7.9901x; 7.9901x over previous
import functools

import jax
import jax.numpy as jnp
from jax import lax
from jax.experimental import pallas as pl
from jax.experimental.pallas import tpu as pltpu
from jax.experimental.pallas import tpu_sc as plsc

BATCH = 16384
EMBED = 32
_INFO = plsc.get_sparse_core_info()
NC, NS, L = _INFO.num_cores, _INFO.num_subcores, _INFO.num_lanes
NW = NC * NS
B_PER_W = BATCH // NW
N_CHUNK = 4
CHUNK = B_PER_W // N_CHUNK
PROJ_BLK = 65536


def _tc_proj_body(w_ref, t_ref, o_ref):
    w = w_ref[...]
    prod = jnp.dot(w, t_ref[...],
                   preferred_element_type=jnp.float32)
    o_ref[...] = prod.reshape(PROJ_BLK)


def _tc_project(table_t, w_row):
    n = table_t.shape[1]
    grid = (n + PROJ_BLK - 1) // PROJ_BLK
    return pl.pallas_call(
        _tc_proj_body,
        grid=(grid,),
        in_specs=[
            pl.BlockSpec((1, EMBED), lambda i: (0, 0)),
            pl.BlockSpec((EMBED, PROJ_BLK), lambda i: (0, i)),
        ],
        out_specs=pl.BlockSpec((PROJ_BLK,), lambda i: (i,)),
        out_shape=jax.ShapeDtypeStruct((n,), jnp.float32),
    )(w_row, table_t)


def _sc_body(users_hbm, autos_hbm, pu_hbm, pa_hbm, b_hbm,
             out_hbm, idx_u, idx_a, gu, ga, out_v, b_v, sem):
    wid = lax.axis_index("s") * NC + lax.axis_index("c")

    pltpu.sync_copy(users_hbm.at[wid], idx_u)
    pltpu.sync_copy(autos_hbm.at[wid], idx_a)
    pltpu.sync_copy(b_hbm, b_v)

    copies = []
    for j in range(N_CHUNK):
        copies.append(pltpu.async_copy(
            pu_hbm.at[idx_u.at[j]], gu.at[pl.ds(j * CHUNK, CHUNK)], sem))
        copies.append(pltpu.async_copy(
            pa_hbm.at[idx_a.at[j]], ga.at[pl.ds(j * CHUNK, CHUNK)], sem))
    for c in copies:
        c.wait()

    bvec = b_v[...]

    def body(k, carry):
        sl = pl.ds(k * L, L)
        out_v[sl] = gu[sl] + ga[sl] + bvec
        return carry

    lax.fori_loop(0, B_PER_W // L, body, 0)

    pltpu.sync_copy(out_v, out_hbm.at[pl.ds(wid * B_PER_W, B_PER_W)])


def _sc_gather(users_r, autos_r, pu, pa, b16):
    mesh = plsc.VectorSubcoreMesh(core_axis_name="c", subcore_axis_name="s")
    f = functools.partial(
        pl.kernel, mesh=mesh,
        compiler_params=pltpu.CompilerParams(needs_layout_passes=False,
                                             use_tc_tiling_on_sc=False),
        out_type=jax.ShapeDtypeStruct((BATCH,), jnp.float32),
        scratch_types=[
            pltpu.VMEM((N_CHUNK, CHUNK), jnp.int32),
            pltpu.VMEM((N_CHUNK, CHUNK), jnp.int32),
            pltpu.VMEM((B_PER_W,), jnp.float32),
            pltpu.VMEM((B_PER_W,), jnp.float32),
            pltpu.VMEM((B_PER_W,), jnp.float32),
            pltpu.VMEM((L,), jnp.float32),
            pltpu.SemaphoreType.DMA,
        ],
    )(_sc_body)
    return f(users_r, autos_r, pu, pa, b16)


@jax.jit
def _run(users_r, autos_r, user_table_t, auto_table_t, W, b):
    wf = W.astype(jnp.float32)
    wu = wf[:EMBED].reshape(1, EMBED)
    wa = wf[EMBED:].reshape(1, EMBED)
    pu = _tc_project(user_table_t, wu)
    pa = _tc_project(auto_table_t, wa)
    b16 = jnp.broadcast_to(b.astype(jnp.float32), (L,))
    return _sc_gather(users_r, autos_r, pu, pa, b16)


def kernel(users, autos, user_table, auto_table, W, b):
    users_r = users.astype(jnp.int32).reshape(NW, N_CHUNK, CHUNK)
    autos_r = autos.astype(jnp.int32).reshape(NW, N_CHUNK, CHUNK)
    out = _run(users_r, autos_r, user_table.T, auto_table.T, W, b)
    return out.reshape(BATCH, 1)
